# initial kernel scaffold (unmeasured)
import jax
import jax.numpy as jnp
from jax import lax
from jax.experimental import pallas as pl
from jax.experimental.pallas import tpu as pltpu

N_DEV = 8


def kernel(x, w_mat, scale_x, scale_w):
    m_glob, k_shard = x.shape
    k_glob, n = w_mat.shape
    m_per = m_glob // N_DEV

    def body(x_ref, w_ref, sx_ref, sw_ref, out_ref, xg_ref, send_sems, recv_sems):
        my = lax.axis_index("i")

        barrier = pltpu.get_barrier_semaphore()
        for off in range(1, N_DEV):
            peer = lax.rem(my + off, N_DEV)
            pl.semaphore_signal(
                barrier, inc=1, device_id=(peer,),
                device_id_type=pl.DeviceIdType.MESH,
            )
        pl.semaphore_wait(barrier, N_DEV - 1)

        xg_ref[:, pl.ds(my * k_shard, k_shard)] = x_ref[pl.ds(my * m_per, m_per), :]

        rdmas = []
        for off in range(1, N_DEV):
            peer = lax.rem(my + off, N_DEV)
            rdma = pltpu.make_async_remote_copy(
                src_ref=x_ref.at[pl.ds(peer * m_per, m_per), :],
                dst_ref=xg_ref.at[:, pl.ds(my * k_shard, k_shard)],
                send_sem=send_sems.at[off - 1],
                recv_sem=recv_sems.at[off - 1],
                device_id=(peer,),
                device_id_type=pl.DeviceIdType.MESH,
            )
            rdma.start()
            rdmas.append(rdma)

        for r in rdmas:
            r.wait_send()
        for r in rdmas:
            r.wait_recv()

        acc = jnp.dot(xg_ref[:, :], w_ref[:, :], preferred_element_type=jnp.int32)
        out_ref[:, :] = acc.astype(jnp.float32) * (sx_ref[0] * sw_ref[0])

    return pl.pallas_call(
        body,
        out_shape=jax.ShapeDtypeStruct((m_per, n), jnp.float32),
        in_specs=[
            pl.BlockSpec(memory_space=pltpu.VMEM),
            pl.BlockSpec(memory_space=pltpu.VMEM),
            pl.BlockSpec(memory_space=pltpu.SMEM),
            pl.BlockSpec(memory_space=pltpu.SMEM),
        ],
        out_specs=pl.BlockSpec(memory_space=pltpu.VMEM),
        scratch_shapes=[
            pltpu.VMEM((m_per, k_glob), jnp.int8),
            pltpu.SemaphoreType.DMA((N_DEV - 1,)),
            pltpu.SemaphoreType.DMA((N_DEV - 1,)),
        ],
        compiler_params=pltpu.CompilerParams(collective_id=0),
    )(x, w_mat, scale_x, scale_w)


# baseline (device time: 87206 ns/iter reference)
import jax
import jax.numpy as jnp
from jax import lax
from jax.experimental import pallas as pl
from jax.experimental.pallas import tpu as pltpu

N_DEV = 8


def kernel(x, w_mat, scale_x, scale_w):
    m_glob, k_shard = x.shape
    k_glob, n = w_mat.shape
    m_per = m_glob // N_DEV

    def body(x_ref, w_ref, sx_ref, sw_ref, out_ref, xg_ref, send_sems, recv_sems):
        my = lax.axis_index("i")

        barrier = pltpu.get_barrier_semaphore()
        for off in range(1, N_DEV):
            peer = lax.rem(my + off, N_DEV)
            pl.semaphore_signal(
                barrier, inc=1, device_id=(peer,),
                device_id_type=pl.DeviceIdType.MESH,
            )
        pl.semaphore_wait(barrier, N_DEV - 1)

        xg_ref[:, pl.ds(my * k_shard, k_shard)] = x_ref[pl.ds(my * m_per, m_per), :]

        rdmas = []
        for off in range(1, N_DEV):
            peer = lax.rem(my + off, N_DEV)
            rdma = pltpu.make_async_remote_copy(
                src_ref=x_ref.at[pl.ds(peer * m_per, m_per), :],
                dst_ref=xg_ref.at[:, pl.ds(my * k_shard, k_shard)],
                send_sem=send_sems.at[off - 1],
                recv_sem=recv_sems.at[off - 1],
                device_id=(peer,),
                device_id_type=pl.DeviceIdType.MESH,
            )
            rdma.start()
            rdmas.append(rdma)

        for r in rdmas:
            r.wait_send()
        for r in rdmas:
            r.wait_recv()

        acc = jnp.dot(xg_ref[:, :], w_ref[:, :], preferred_element_type=jnp.int32)
        out_ref[:, :] = acc.astype(jnp.float32) * (sx_ref[0] * sw_ref[0])

    return pl.pallas_call(
        body,
        out_shape=jax.ShapeDtypeStruct((m_per, n), jnp.float32),
        in_specs=[
            pl.BlockSpec(memory_space=pltpu.VMEM),
            pl.BlockSpec(memory_space=pltpu.VMEM),
            pl.BlockSpec(memory_space=pltpu.SMEM),
            pl.BlockSpec(memory_space=pltpu.SMEM),
        ],
        out_specs=pl.BlockSpec(memory_space=pltpu.VMEM),
        scratch_shapes=[
            pltpu.VMEM((m_per, k_glob), jnp.int8),
            pltpu.SemaphoreType.DMA((N_DEV - 1,)),
            pltpu.SemaphoreType.DMA((N_DEV - 1,)),
        ],
        compiler_params=pltpu.CompilerParams(
            collective_id=0,
            vmem_limit_bytes=100 * 1024 * 1024,
        ),
    )(x, w_mat, scale_x, scale_w)


# device time: 72980 ns/iter; 1.1949x vs baseline; 1.1949x over previous
import jax
import jax.numpy as jnp
from jax import lax
from jax.experimental import pallas as pl
from jax.experimental.pallas import tpu as pltpu

N_DEV = 8
M_PER = 512
K_SHARD = 512
K_GLOB = 4096
N_GLOB = 8192
NC = 2048
KG = 2
K_GRP = K_GLOB // KG
NJ = N_GLOB // NC


def kernel(x, w_mat, scale_x, scale_w):
    def body(x_ref, w_ref, sx_ref, sw_ref, out_ref,
             xg8_ref, xgf_ref, send_sems, recv_sems):
        nj = pl.program_id(0)
        kg = pl.program_id(1)
        my = lax.axis_index("i")
        my_half = my // 4

        @pl.when(jnp.logical_and(nj == 0, kg == 0))
        def _():
            barrier = pltpu.get_barrier_semaphore()
            for off in range(1, N_DEV):
                peer = lax.rem(my + off, N_DEV)
                pl.semaphore_signal(
                    barrier, inc=1, device_id=(peer,),
                    device_id_type=pl.DeviceIdType.MESH,
                )
            pl.semaphore_wait(barrier, N_DEV - 1)

            base = my - lax.rem(my, 4)
            peers = []
            for t in range(1, 4):
                peers.append(base + lax.rem(my - base + t, 4))
            opp = lax.rem(base + 4, N_DEV)
            for t in range(4):
                peers.append(opp + lax.rem(my - base + t, 4))
            for k, peer in enumerate(peers):
                rdma = pltpu.make_async_remote_copy(
                    src_ref=x_ref.at[pl.ds(peer * M_PER, M_PER), :],
                    dst_ref=xg8_ref.at[my],
                    send_sem=send_sems.at[k],
                    recv_sem=recv_sems.at[my],
                    device_id=(peer,),
                    device_id_type=pl.DeviceIdType.MESH,
                )
                rdma.start()

            xgf_ref[:, pl.ds(my * K_SHARD, K_SHARD)] = (
                x_ref[pl.ds(my * M_PER, M_PER), :])

        grp = lax.rem(my_half + kg, KG)

        @pl.when(nj == 0)
        def _():
            for t in range(4):
                s = 4 * grp + t

                @pl.when(s != my)
                def _():
                    recv = pltpu.make_async_remote_copy(
                        src_ref=x_ref.at[pl.ds(0, M_PER), :],
                        dst_ref=xg8_ref.at[s],
                        send_sem=send_sems.at[0],
                        recv_sem=recv_sems.at[s],
                        device_id=(my,),
                        device_id_type=pl.DeviceIdType.MESH,
                    )
                    recv.wait_recv()
                    xgf_ref[:, pl.ds(s * K_SHARD, K_SHARD)] = xg8_ref[s]

        partial = jax.lax.dot_general(
            xgf_ref[:, pl.ds(grp * K_GRP, K_GRP)], w_ref[:, :],
            (((1,), (0,)), ((), ())),
            preferred_element_type=jnp.int32,
        )

        @pl.when(kg == 0)
        def _():
            out_ref[:, :] = partial.astype(jnp.float32)

        @pl.when(kg == 1)
        def _():
            out_ref[:, :] = (
                (out_ref[:, :] + partial.astype(jnp.float32))
                * (sx_ref[0] * sw_ref[0]))

        @pl.when(jnp.logical_and(nj == NJ - 1, kg == KG - 1))
        def _():
            for k in range(N_DEV - 1):
                done = pltpu.make_async_remote_copy(
                    src_ref=x_ref.at[pl.ds(0, M_PER), :],
                    dst_ref=xg8_ref.at[0],
                    send_sem=send_sems.at[k],
                    recv_sem=recv_sems.at[my],
                    device_id=(my,),
                    device_id_type=pl.DeviceIdType.MESH,
                )
                done.wait_send()

    grid = (NJ, KG)
    return pl.pallas_call(
        body,
        grid=grid,
        out_shape=jax.ShapeDtypeStruct((M_PER, N_GLOB), jnp.float32),
        in_specs=[
            pl.BlockSpec((K_GLOB, K_SHARD), lambda nj, kg: (0, 0)),
            pl.BlockSpec(
                (K_GRP, NC),
                lambda nj, kg: (lax.rem(lax.axis_index("i") // 4 + kg, KG), nj),
            ),
            pl.BlockSpec(memory_space=pltpu.SMEM),
            pl.BlockSpec(memory_space=pltpu.SMEM),
        ],
        out_specs=pl.BlockSpec((M_PER, NC), lambda nj, kg: (0, nj)),
        scratch_shapes=[
            pltpu.VMEM((N_DEV, M_PER, K_SHARD), jnp.int8),
            pltpu.VMEM((M_PER, K_GLOB), jnp.int8),
            pltpu.SemaphoreType.DMA((N_DEV - 1,)),
            pltpu.SemaphoreType.DMA((N_DEV,)),
        ],
        compiler_params=pltpu.CompilerParams(
            collective_id=0,
            vmem_limit_bytes=100 * 1024 * 1024,
        ),
    )(x, w_mat, scale_x, scale_w)
